# baseline (speedup 1.0000x reference)
import functools
import jax
import jax.numpy as jnp
from jax import lax
from jax.experimental import pallas as pl
from jax.experimental.pallas import tpu as pltpu
from jax.experimental.pallas import tpu_sc as plsc

OBS_VOCAB = 1000000
ACT_VOCAB = 1000
D = 16
BATCH = 16384
HIST = 50

V = OBS_VOCAB
B = BATCH
H = HIST
HP = 64
NC, NS = 2, 16
CW = 1024
NCH = 488
HALF0 = NCH * CW
TAIL = V - HALF0 - NCH * CW
TAILP = 640
SPROJ = NCH * CW + TAILP + 16
ZSLOT = NCH * CW + TAILP
RW = B // NS
NQ = 8
QRW = RW // NQ
QIDX = QRW * HP


def _sc_body(tblT_hbm, tail_hbm, obs_hbm, w1_hbm, out_hbm,
             tchunk_v, acc_v, w1_v, idx_v, vals_v, enc_v, sproj, sem):
    cid = lax.axis_index("c")
    sid = lax.axis_index("s")

    pltpu.sync_copy(w1_hbm, w1_v)
    w1vec = w1_v[...]
    lanes = lax.iota(jnp.int32, 16)
    w1s = [jnp.sum(jnp.where(lanes == k, w1vec, 0.0)) for k in range(D)]

    half_base = cid * HALF0
    half_size = jnp.where(cid == 0, HALF0, V - HALF0)

    def col_reduce(ncols):
        def col_body(q, carry2):
            col = q * 16
            a = w1s[0] * tchunk_v[0, pl.ds(col, 16)]
            for k in range(1, D):
                a = a + w1s[k] * tchunk_v[k, pl.ds(col, 16)]
            acc_v[pl.ds(col, 16)] = a
            return carry2

        lax.fori_loop(0, ncols // 16, col_body, 0)

    def chunk_body(j, carry):
        c = j * NS + sid

        @pl.when(c < NCH)
        def _():
            g0 = pl.multiple_of(half_base + c * CW, 128)
            pltpu.sync_copy(tblT_hbm.at[:, pl.ds(g0, CW)], tchunk_v)
            col_reduce(CW)
            pltpu.sync_copy(acc_v, sproj.at[pl.ds(c * CW, CW)])

        return carry

    lax.fori_loop(0, -(-NCH // NS), chunk_body, 0)

    @pl.when((sid == 0) & (cid == 1))
    def _():
        pltpu.sync_copy(tail_hbm, tchunk_v.at[:, pl.ds(0, TAILP)])
        col_reduce(TAILP)
        pltpu.sync_copy(acc_v.at[pl.ds(0, TAILP)],
                        sproj.at[pl.ds(NCH * CW, TAILP)])

    @pl.when(sid == 1)
    def _():
        acc_v[pl.ds(0, 16)] = jnp.zeros((16,), jnp.float32)
        pltpu.sync_copy(acc_v.at[pl.ds(0, 16)], sproj.at[pl.ds(ZSLOT, 16)])

    plsc.subcore_barrier()

    base = sid * RW
    tail_mask = lanes < (H - 48)

    def quarter_body(hq, carry):
        r0 = base + hq * QRW
        pltpu.sync_copy(obs_hbm.at[pl.ds(r0 * HP, QIDX)], idx_v)

        def fix_body(q, carry2):
            col = q * 16
            raw = idx_v[pl.ds(col, 16)] - half_base
            ok = (raw >= 0) & (raw < half_size)
            idx_v[pl.ds(col, 16)] = jnp.where(ok, raw, ZSLOT + lanes)
            return carry2

        lax.fori_loop(0, QIDX // 16, fix_body, 0)
        pltpu.async_copy(sproj.at[idx_v], vals_v, sem).wait()

        def row_body(r, carry2):
            rb = r * HP
            v = (vals_v[pl.ds(rb, 16)] + vals_v[pl.ds(rb + 16, 16)]
                 + vals_v[pl.ds(rb + 32, 16)]
                 + jnp.where(tail_mask, vals_v[pl.ds(rb + 48, 16)], 0.0))
            enc_v[r] = v
            return carry2

        lax.fori_loop(0, QRW, row_body, 0)
        pltpu.sync_copy(enc_v, out_hbm.at[cid, pl.ds(r0, QRW), :])
        return carry

    lax.fori_loop(0, NQ, quarter_body, 0)


@functools.cache
def _bag():
    return pl.kernel(
        _sc_body,
        out_type=jax.ShapeDtypeStruct((NC, B, D), jnp.float32),
        mesh=plsc.VectorSubcoreMesh(core_axis_name="c", subcore_axis_name="s"),
        scratch_types=[
            pltpu.VMEM((D, CW), jnp.float32),
            pltpu.VMEM((CW,), jnp.float32),
            pltpu.VMEM((16,), jnp.float32),
            pltpu.VMEM((QIDX,), jnp.int32),
            pltpu.VMEM((QIDX,), jnp.float32),
            pltpu.VMEM((QRW, D), jnp.float32),
            pltpu.VMEM_SHARED((SPROJ,), jnp.float32),
            pltpu.SemaphoreType.DMA,
        ],
        compiler_params=pltpu.CompilerParams(use_tc_tiling_on_sc=True,
                                             needs_layout_passes=False),
    )


RB_ACT = 512


def _act_body(a_ref, tbl_ref, w_ref, b_ref, out_ref):
    a = a_ref[...]
    m = jnp.max(a, axis=1, keepdims=True)
    iota = lax.broadcasted_iota(jnp.int32, a.shape, 1)
    idx = jnp.min(jnp.where(a == m, iota, ACT_VOCAB), axis=1, keepdims=True)
    onehot = (iota == idx).astype(jnp.float32)
    emb = jnp.dot(onehot, tbl_ref[...], preferred_element_type=jnp.float32)
    w2 = w_ref[:, D:]
    r = jnp.sum(emb * w2, axis=1, keepdims=True)
    out_ref[...] = r + b_ref[0, 0]


def _act_part(actions, action_table, W, b2d):
    return pl.pallas_call(
        _act_body,
        grid=(BATCH // RB_ACT,),
        in_specs=[
            pl.BlockSpec((RB_ACT, ACT_VOCAB), lambda i: (i, 0)),
            pl.BlockSpec((ACT_VOCAB, D), lambda i: (0, 0)),
            pl.BlockSpec((1, 2 * D), lambda i: (0, 0)),
            pl.BlockSpec((1, 1), lambda i: (0, 0)),
        ],
        out_specs=pl.BlockSpec((RB_ACT, 1), lambda i: (i, 0)),
        out_shape=jax.ShapeDtypeStruct((BATCH, 1), jnp.float32),
    )(actions, action_table, W, b2d)


RB_COMB = 2048


def _comb_body(p_ref, act_ref, out_ref):
    s = p_ref[0] + p_ref[1]
    out_ref[...] = jnp.sum(s, axis=1, keepdims=True) + act_ref[...]


def _combine(parts, act_part):
    return pl.pallas_call(
        _comb_body,
        grid=(BATCH // RB_COMB,),
        in_specs=[
            pl.BlockSpec((NC, RB_COMB, D), lambda i: (0, i, 0)),
            pl.BlockSpec((RB_COMB, 1), lambda i: (i, 0)),
        ],
        out_specs=pl.BlockSpec((RB_COMB, 1), lambda i: (i, 0)),
        out_shape=jax.ShapeDtypeStruct((BATCH, 1), jnp.float32),
    )(parts, act_part)


def kernel(observation, actions, obs_table, action_table, W, b):
    obs_pad = jnp.pad(observation.astype(jnp.int32), ((0, 0), (0, HP - H)))
    obs_flat = obs_pad.reshape(-1)
    w1s = W[0, :D] * (1.0 / H)
    tblT = obs_table.T
    tail = jnp.pad(lax.slice(tblT, (0, V - 576), (D, V)),
                   ((0, 0), (0, TAILP - 576)))
    parts = _bag()(tblT, tail, obs_flat, w1s)
    act = _act_part(actions, action_table, W,
                    b.reshape(1, 1).astype(jnp.float32))
    return _combine(parts, act)


# speedup vs baseline: 1.8858x; 1.8858x over previous
import functools
import jax
import jax.numpy as jnp
from jax import lax
from jax.experimental import pallas as pl
from jax.experimental.pallas import tpu as pltpu
from jax.experimental.pallas import tpu_sc as plsc

OBS_VOCAB = 1000000
ACT_VOCAB = 1000
D = 16
BATCH = 16384
HIST = 50

V = OBS_VOCAB
B = BATCH
H = HIST
NC, NS = 2, 16
CW = 512
NCH = 976
NCHT = NCH // NS
HALF0 = NCH * CW
TAIL = V - 2 * HALF0
TAILP = 640
ZREG = 128
SPROJ = NCH * CW + TAILP + ZREG
ZSLOT = NCH * CW + TAILP
RW = B // NS
NQ = 8
QRW = RW // NQ
QIDX = QRW * H


def _sc_body(tblT_hbm, tail_hbm, obs_hbm, w1_hbm, out_hbm,
             tch0, tch1, acc_v, w1_v, idx0, idx1, val0, val1, enc_v, sproj,
             sem0, sem1, gsem0, gsem1):
    cid = lax.axis_index("c")
    sid = lax.axis_index("s")

    pltpu.sync_copy(w1_hbm, w1_v)
    w1vec = w1_v[...]
    lanes = lax.iota(jnp.int32, 16)
    w1s = [jnp.sum(jnp.where(lanes == k, w1vec, 0.0)) for k in range(D)]

    half_base = cid * HALF0
    half_size = jnp.where(cid == 0, HALF0, V - HALF0)

    tch = (tch0, tch1)
    sems = (sem0, sem1)

    def src_slice(j):
        c = j * NS + sid
        g0 = pl.multiple_of(half_base + c * CW, 128)
        return tblT_hbm.at[:, pl.ds(g0, CW)], c

    def p1_start(j, buf):
        @pl.when(j < NCHT)
        def _():
            src, _ = src_slice(j)
            pltpu.async_copy(src, tch[buf], sems[buf])

    def col_reduce(buf_ref, ncols):
        def col_body(q, carry2):
            col = q * 16
            a = w1s[0] * buf_ref[0, pl.ds(col, 16)]
            for k in range(1, D):
                a = a + w1s[k] * buf_ref[k, pl.ds(col, 16)]
            acc_v[pl.ds(col, 16)] = a
            return carry2

        lax.fori_loop(0, ncols // 16, col_body, 0)

    def p1_finish(j, buf):
        @pl.when(j < NCHT)
        def _():
            src, c = src_slice(j)
            pltpu.make_async_copy(src, tch[buf], sems[buf]).wait()
            col_reduce(tch[buf], CW)
            pltpu.sync_copy(acc_v.at[pl.ds(0, CW)], sproj.at[pl.ds(c * CW, CW)])

    p1_start(0, 0)

    def p1_pair(m, carry):
        j0 = m * 2
        p1_start(j0 + 1, 1)
        p1_finish(j0, 0)
        p1_start(j0 + 2, 0)
        p1_finish(j0 + 1, 1)
        return carry

    lax.fori_loop(0, (NCHT + 1) // 2, p1_pair, 0)

    @pl.when((sid == 0) & (cid == 1))
    def _():
        pltpu.sync_copy(tail_hbm, tch0.at[:, pl.ds(0, TAILP)])
        col_reduce(tch0, TAILP)
        pltpu.sync_copy(acc_v.at[pl.ds(0, TAILP)],
                        sproj.at[pl.ds(NCH * CW, TAILP)])

    @pl.when(sid == 1)
    def _():
        zeros = jnp.zeros((16,), jnp.float32)
        for z in range(ZREG // 16):
            acc_v[pl.ds(z * 16, 16)] = zeros
        pltpu.sync_copy(acc_v.at[pl.ds(0, ZREG)], sproj.at[pl.ds(ZSLOT, ZREG)])

    plsc.subcore_barrier()

    base = sid * RW
    tail_mask = lanes < (H - 48)
    idxb = (idx0, idx1)
    valb = (val0, val1)
    gsems = (gsem0, gsem1)

    def q_start(hq, buf):
        r0 = base + hq * QRW
        pltpu.sync_copy(obs_hbm.at[pl.ds(r0 * H, QIDX)], idxb[buf])

        def fix_body(q, carry2):
            col = q * 16
            raw = idxb[buf][pl.ds(col, 16)] - half_base
            ok = (raw >= 0) & (raw < half_size)
            dummy = ZSLOT + ((col & (ZREG - 16)) | lanes)
            idxb[buf][pl.ds(col, 16)] = jnp.where(ok, raw, dummy)
            return carry2

        lax.fori_loop(0, QIDX // 16, fix_body, 0)
        pltpu.async_copy(sproj.at[idxb[buf]], valb[buf].at[pl.ds(0, QIDX)],
                         gsems[buf])

    def q_finish(hq, buf):
        r0 = base + hq * QRW
        pltpu.make_async_copy(sproj.at[idxb[buf]], valb[buf].at[pl.ds(0, QIDX)],
                              gsems[buf]).wait()
        vv = valb[buf]

        def row_body(r, carry2):
            rb = r * H
            v = (vv[pl.ds(rb, 16)] + vv[pl.ds(rb + 16, 16)]
                 + vv[pl.ds(rb + 32, 16)]
                 + jnp.where(tail_mask, vv[pl.ds(rb + 48, 16)], 0.0))
            enc_v[r] = v
            return carry2

        lax.fori_loop(0, QRW, row_body, 0)
        pltpu.sync_copy(enc_v, out_hbm.at[cid, pl.ds(r0, QRW), :])

    q_start(0, 0)
    for hq in range(NQ):
        if hq + 1 < NQ:
            q_start(hq + 1, (hq + 1) % 2)
        q_finish(hq, hq % 2)


@functools.cache
def _bag():
    return pl.kernel(
        _sc_body,
        out_type=jax.ShapeDtypeStruct((NC, B, D), jnp.float32),
        mesh=plsc.VectorSubcoreMesh(core_axis_name="c", subcore_axis_name="s"),
        scratch_types=[
            pltpu.VMEM((D, CW), jnp.float32),
            pltpu.VMEM((D, CW), jnp.float32),
            pltpu.VMEM((TAILP,), jnp.float32),
            pltpu.VMEM((16,), jnp.float32),
            pltpu.VMEM((QIDX,), jnp.int32),
            pltpu.VMEM((QIDX,), jnp.int32),
            pltpu.VMEM((QIDX + 16,), jnp.float32),
            pltpu.VMEM((QIDX + 16,), jnp.float32),
            pltpu.VMEM((QRW, D), jnp.float32),
            pltpu.VMEM_SHARED((SPROJ,), jnp.float32),
            pltpu.SemaphoreType.DMA,
            pltpu.SemaphoreType.DMA,
            pltpu.SemaphoreType.DMA,
            pltpu.SemaphoreType.DMA,
        ],
        compiler_params=pltpu.CompilerParams(use_tc_tiling_on_sc=True,
                                             needs_layout_passes=False),
    )


RB_ACT = 512


def _act_body(a_ref, tbl_ref, w_ref, b_ref, out_ref):
    a = a_ref[...]
    m = jnp.max(a, axis=1, keepdims=True)
    iota = lax.broadcasted_iota(jnp.int32, a.shape, 1)
    idx = jnp.min(jnp.where(a == m, iota, ACT_VOCAB), axis=1, keepdims=True)
    onehot = (iota == idx).astype(jnp.float32)
    emb = jnp.dot(onehot, tbl_ref[...], preferred_element_type=jnp.float32)
    w2 = w_ref[:, D:]
    r = jnp.sum(emb * w2, axis=1, keepdims=True)
    out_ref[...] = r + b_ref[0, 0]


def _act_part(actions, action_table, W, b2d):
    return pl.pallas_call(
        _act_body,
        grid=(BATCH // RB_ACT,),
        in_specs=[
            pl.BlockSpec((RB_ACT, ACT_VOCAB), lambda i: (i, 0)),
            pl.BlockSpec((ACT_VOCAB, D), lambda i: (0, 0)),
            pl.BlockSpec((1, 2 * D), lambda i: (0, 0)),
            pl.BlockSpec((1, 1), lambda i: (0, 0)),
        ],
        out_specs=pl.BlockSpec((RB_ACT, 1), lambda i: (i, 0)),
        out_shape=jax.ShapeDtypeStruct((BATCH, 1), jnp.float32),
    )(actions, action_table, W, b2d)


RB_COMB = 2048


def _comb_body(p_ref, act_ref, out_ref):
    s = p_ref[0] + p_ref[1]
    out_ref[...] = jnp.sum(s, axis=1, keepdims=True) + act_ref[...]


def _combine(parts, act_part):
    return pl.pallas_call(
        _comb_body,
        grid=(BATCH // RB_COMB,),
        in_specs=[
            pl.BlockSpec((NC, RB_COMB, D), lambda i: (0, i, 0)),
            pl.BlockSpec((RB_COMB, 1), lambda i: (i, 0)),
        ],
        out_specs=pl.BlockSpec((RB_COMB, 1), lambda i: (i, 0)),
        out_shape=jax.ShapeDtypeStruct((BATCH, 1), jnp.float32),
    )(parts, act_part)


def kernel(observation, actions, obs_table, action_table, W, b):
    obs_flat = observation.astype(jnp.int32).reshape(-1)
    w1s = W[0, :D] * (1.0 / H)
    tblT = obs_table.T
    tail = jnp.pad(lax.slice(tblT, (0, V - TAIL), (D, V)),
                   ((0, 0), (0, TAILP - TAIL)))
    parts = _bag()(tblT, tail, obs_flat, w1s)
    act = _act_part(actions, action_table, W,
                    b.reshape(1, 1).astype(jnp.float32))
    return _combine(parts, act)


# speedup vs baseline: 2.3426x; 1.2422x over previous
import functools
import jax
import jax.numpy as jnp
from jax import lax
from jax.experimental import pallas as pl
from jax.experimental.pallas import tpu as pltpu
from jax.experimental.pallas import tpu_sc as plsc

OBS_VOCAB = 1000000
ACT_VOCAB = 1000
D = 16
BATCH = 16384
HIST = 50

V = OBS_VOCAB
B = BATCH
H = HIST
NC, NS = 2, 16
CW = 512
NCH = 976
NCHT = NCH // NS
HALF0 = NCH * CW
TAIL = V - 2 * HALF0
TAILP = 640
ZREG = 128
SPROJ = NCH * CW + TAILP + ZREG
ZSLOT = NCH * CW + TAILP
RW = B // NS
NQ = 8
QRW = RW // NQ
QIDX = QRW * H
CB8 = QRW // 16


def _sc_body(tblT_hbm, tail_hbm, obsT_hbm, w1_hbm, out_hbm,
             tch0, tch1, acc_v, w1_v, idx0, idx1, val0, val1, enc_v, sproj,
             sem0, sem1, gsem0, gsem1):
    cid = lax.axis_index("c")
    sid = lax.axis_index("s")

    pltpu.sync_copy(w1_hbm, w1_v)
    w1vec = w1_v[...]
    lanes = lax.iota(jnp.int32, 16)
    w1s = [jnp.sum(jnp.where(lanes == k, w1vec, 0.0)) for k in range(D)]

    half_base = cid * HALF0
    half_size = jnp.where(cid == 0, HALF0, V - HALF0)

    tch = (tch0, tch1)
    sems = (sem0, sem1)

    def src_slice(j):
        c = j * NS + sid
        g0 = pl.multiple_of(half_base + c * CW, 128)
        return tblT_hbm.at[:, pl.ds(g0, CW)], c

    def p1_start(j, buf):
        @pl.when(j < NCHT)
        def _():
            src, _ = src_slice(j)
            pltpu.async_copy(src, tch[buf], sems[buf])

    def col_reduce(buf_ref, ncols):
        def col_body(q, carry2):
            col = q * 16
            a = w1s[0] * buf_ref[0, pl.ds(col, 16)]
            for k in range(1, D):
                a = a + w1s[k] * buf_ref[k, pl.ds(col, 16)]
            acc_v[pl.ds(col, 16)] = a
            return carry2

        lax.fori_loop(0, ncols // 16, col_body, 0)

    def p1_finish(j, buf):
        @pl.when(j < NCHT)
        def _():
            src, c = src_slice(j)
            pltpu.make_async_copy(src, tch[buf], sems[buf]).wait()
            col_reduce(tch[buf], CW)
            pltpu.sync_copy(acc_v.at[pl.ds(0, CW)], sproj.at[pl.ds(c * CW, CW)])

    p1_start(0, 0)

    def p1_pair(m, carry):
        j0 = m * 2
        p1_start(j0 + 1, 1)
        p1_finish(j0, 0)
        p1_start(j0 + 2, 0)
        p1_finish(j0 + 1, 1)
        return carry

    lax.fori_loop(0, (NCHT + 1) // 2, p1_pair, 0)

    @pl.when((sid == 0) & (cid == 1))
    def _():
        pltpu.sync_copy(tail_hbm, tch0.at[:, pl.ds(0, TAILP)])
        col_reduce(tch0, TAILP)
        pltpu.sync_copy(acc_v.at[pl.ds(0, TAILP)],
                        sproj.at[pl.ds(NCH * CW, TAILP)])

    @pl.when(sid == 1)
    def _():
        zeros = jnp.zeros((16,), jnp.float32)
        for z in range(ZREG // 16):
            acc_v[pl.ds(z * 16, 16)] = zeros
        pltpu.sync_copy(acc_v.at[pl.ds(0, ZREG)], sproj.at[pl.ds(ZSLOT, ZREG)])

    plsc.subcore_barrier()

    base = sid * RW
    idxb = (idx0, idx1)
    valb = (val0, val1)
    gsems = (gsem0, gsem1)

    def q_start(hq, buf):
        r0 = base + hq * QRW
        pltpu.sync_copy(obsT_hbm.at[:, pl.ds(r0, QRW)], idxb[buf])

        def fix_body(h, carry2):
            for cb in range(CB8):
                col = cb * 16
                raw = idxb[buf][h, pl.ds(col, 16)] - half_base
                ok = (raw >= 0) & (raw < half_size)
                dummy = ZSLOT + ((col & (ZREG - 16)) | lanes)
                idxb[buf][h, pl.ds(col, 16)] = jnp.where(ok, raw, dummy)
            return carry2

        lax.fori_loop(0, H, fix_body, 0)

        def fire(h, carry2):
            pltpu.async_copy(sproj.at[idxb[buf].at[h]], valb[buf].at[h],
                             gsems[buf])
            return carry2

        lax.fori_loop(0, H, fire, 0)

    def q_finish(hq, buf):
        r0 = base + hq * QRW

        def drain(h, carry2):
            pltpu.make_async_copy(sproj.at[idxb[buf].at[h]], valb[buf].at[h],
                                  gsems[buf]).wait()
            return carry2

        lax.fori_loop(0, H, drain, 0)
        vv = valb[buf]

        def row_body(h, accs):
            return tuple(accs[cb] + vv[h, pl.ds(cb * 16, 16)]
                         for cb in range(CB8))

        accs = tuple(vv[0, pl.ds(cb * 16, 16)] for cb in range(CB8))
        accs = lax.fori_loop(1, H, row_body, accs)
        for cb in range(CB8):
            enc_v[pl.ds(cb * 16, 16)] = accs[cb]
        pltpu.sync_copy(enc_v, out_hbm.at[cid, pl.ds(r0, QRW)])

    q_start(0, 0)
    for hq in range(NQ):
        if hq + 1 < NQ:
            q_start(hq + 1, (hq + 1) % 2)
        q_finish(hq, hq % 2)


@functools.cache
def _bag():
    return pl.kernel(
        _sc_body,
        out_type=jax.ShapeDtypeStruct((NC, B), jnp.float32),
        mesh=plsc.VectorSubcoreMesh(core_axis_name="c", subcore_axis_name="s"),
        scratch_types=[
            pltpu.VMEM((D, CW), jnp.float32),
            pltpu.VMEM((D, CW), jnp.float32),
            pltpu.VMEM((TAILP,), jnp.float32),
            pltpu.VMEM((16,), jnp.float32),
            pltpu.VMEM((H, QRW), jnp.int32),
            pltpu.VMEM((H, QRW), jnp.int32),
            pltpu.VMEM((H, QRW), jnp.float32),
            pltpu.VMEM((H, QRW), jnp.float32),
            pltpu.VMEM((QRW,), jnp.float32),
            pltpu.VMEM_SHARED((SPROJ,), jnp.float32),
            pltpu.SemaphoreType.DMA,
            pltpu.SemaphoreType.DMA,
            pltpu.SemaphoreType.DMA,
            pltpu.SemaphoreType.DMA,
        ],
        compiler_params=pltpu.CompilerParams(use_tc_tiling_on_sc=True,
                                             needs_layout_passes=False),
    )


RB_ACT = 512


def _act_body(a_ref, tbl_ref, w_ref, b_ref, out_ref):
    a = a_ref[...]
    m = jnp.max(a, axis=1, keepdims=True)
    iota = lax.broadcasted_iota(jnp.int32, a.shape, 1)
    idx = jnp.min(jnp.where(a == m, iota, ACT_VOCAB), axis=1, keepdims=True)
    onehot = (iota == idx).astype(jnp.float32)
    emb = jnp.dot(onehot, tbl_ref[...], preferred_element_type=jnp.float32)
    w2 = w_ref[:, D:]
    r = jnp.sum(emb * w2, axis=1, keepdims=True)
    out_ref[...] = r + b_ref[0, 0]


def _act_part(actions, action_table, W, b2d):
    return pl.pallas_call(
        _act_body,
        grid=(BATCH // RB_ACT,),
        in_specs=[
            pl.BlockSpec((RB_ACT, ACT_VOCAB), lambda i: (i, 0)),
            pl.BlockSpec((ACT_VOCAB, D), lambda i: (0, 0)),
            pl.BlockSpec((1, 2 * D), lambda i: (0, 0)),
            pl.BlockSpec((1, 1), lambda i: (0, 0)),
        ],
        out_specs=pl.BlockSpec((RB_ACT, 1), lambda i: (i, 0)),
        out_shape=jax.ShapeDtypeStruct((BATCH, 1), jnp.float32),
    )(actions, action_table, W, b2d)


RB_COMB = 2048


def _comb_body(p_ref, act_ref, out_ref):
    s = p_ref[0:1, :] + p_ref[1:2, :]
    out_ref[...] = s + act_ref[...]


def _combine(parts, act_part):
    out2d = pl.pallas_call(
        _comb_body,
        grid=(BATCH // RB_COMB,),
        in_specs=[
            pl.BlockSpec((NC, RB_COMB), lambda i: (0, i)),
            pl.BlockSpec((1, RB_COMB), lambda i: (0, i)),
        ],
        out_specs=pl.BlockSpec((1, RB_COMB), lambda i: (0, i)),
        out_shape=jax.ShapeDtypeStruct((1, BATCH), jnp.float32),
    )(parts, act_part)
    return out2d.reshape(BATCH, 1)


def kernel(observation, actions, obs_table, action_table, W, b):
    obsT = observation.astype(jnp.int32).T
    w1s = W[0, :D] * (1.0 / H)
    tblT = obs_table.T
    tail = jnp.pad(lax.slice(tblT, (0, V - TAIL), (D, V)),
                   ((0, 0), (0, TAILP - TAIL)))
    parts = _bag()(tblT, tail, obsT, w1s)
    act = _act_part(actions, action_table, W,
                    b.reshape(1, 1).astype(jnp.float32))
    return _combine(parts, act.reshape(1, BATCH))
